# baseline (device time: 7466 ns/iter reference)
import jax
import jax.numpy as jnp
from jax import lax
from jax.experimental import pallas as pl
from jax.experimental.pallas import tpu as pltpu

N_DEV = 4
N_PHASE = 2


def kernel(x):
    m_rows, n_cols = x.shape
    rows_p = m_rows // N_PHASE

    def body(x_ref, out_ref, comm_ref, send_sems, recv_sems):
        my = lax.axis_index("i")

        barrier_sem = pltpu.get_barrier_semaphore()
        for k in range(1, N_DEV):
            pl.semaphore_signal(
                barrier_sem, inc=1,
                device_id=((my + k) % N_DEV,),
                device_id_type=pl.DeviceIdType.MESH,
            )

        def local_stats(p):
            xv = x_ref[pl.ds(p * rows_p, rows_p), :]
            m = jnp.max(xv, axis=1, keepdims=True)
            e = jnp.exp(xv - m)
            s = jnp.sum(e, axis=1, keepdims=True)
            out_ref[pl.ds(p * rows_p, rows_p), :] = e
            comm_ref[p, 0, :, :] = jnp.transpose(
                jnp.concatenate([m, s], axis=1)
            )

        def start_sends(p):
            rdmas = []
            for k in range(1, N_DEV):
                rdma = pltpu.make_async_remote_copy(
                    src_ref=comm_ref.at[p, 0],
                    dst_ref=comm_ref.at[p, k],
                    send_sem=send_sems.at[p, k - 1],
                    recv_sem=recv_sems.at[p, k - 1],
                    device_id=((my + k) % N_DEV,),
                    device_id_type=pl.DeviceIdType.MESH,
                )
                rdma.start()
                rdmas.append(rdma)
            return rdmas

        def combine_and_scale(p):
            stats = comm_ref[p, :, :, :]
            m_all = stats[:, 0, :]
            s_all = stats[:, 1, :]
            gmax = jnp.max(m_all, axis=0, keepdims=True)
            gsum = jnp.sum(s_all * jnp.exp(m_all - gmax), axis=0,
                           keepdims=True)
            scale_t = jnp.exp(m_all[0:1, :] - gmax) / gsum
            scale = jnp.transpose(scale_t)
            rs = pl.ds(p * rows_p, rows_p)
            out_ref[rs, :] = out_ref[rs, :] * scale

        local_stats(0)
        pl.semaphore_wait(barrier_sem, N_DEV - 1)
        rdmas0 = start_sends(0)
        local_stats(1)
        rdmas1 = start_sends(1)
        for rdma in rdmas0:
            rdma.wait_recv()
        combine_and_scale(0)
        for rdma in rdmas1:
            rdma.wait_recv()
        combine_and_scale(1)
        for rdma in rdmas0 + rdmas1:
            rdma.wait_send()

    return pl.pallas_call(
        body,
        out_shape=jax.ShapeDtypeStruct((m_rows, n_cols), jnp.float32),
        in_specs=[pl.BlockSpec(memory_space=pltpu.VMEM)],
        out_specs=pl.BlockSpec(memory_space=pltpu.VMEM),
        scratch_shapes=[
            pltpu.VMEM((N_PHASE, N_DEV, 2, rows_p), jnp.float32),
            pltpu.SemaphoreType.DMA((N_PHASE, N_DEV - 1)),
            pltpu.SemaphoreType.DMA((N_PHASE, N_DEV - 1)),
        ],
        compiler_params=pltpu.CompilerParams(collective_id=0),
    )(x)


# device time: 7148 ns/iter; 1.0445x vs baseline; 1.0445x over previous
import jax
import jax.numpy as jnp
from jax import lax
from jax.experimental import pallas as pl
from jax.experimental.pallas import tpu as pltpu

N_DEV = 4


def kernel(x):
    m_rows, n_cols = x.shape

    def body(x_ref, out_ref, comm_ref, send_sems, recv_sems):
        my = lax.axis_index("i")

        barrier_sem = pltpu.get_barrier_semaphore()
        for k in range(1, N_DEV):
            pl.semaphore_signal(
                barrier_sem, inc=1,
                device_id=((my + k) % N_DEV,),
                device_id_type=pl.DeviceIdType.MESH,
            )

        xv = x_ref[:, :]
        m = jnp.max(xv, axis=1, keepdims=True)
        s = jnp.sum(jnp.exp(xv - m), axis=1, keepdims=True)
        comm_ref[0, :, :] = jnp.transpose(
            jnp.concatenate([m, s], axis=1)
        )

        pl.semaphore_wait(barrier_sem, N_DEV - 1)

        rdmas = []
        for k in range(1, N_DEV):
            rdma = pltpu.make_async_remote_copy(
                src_ref=comm_ref.at[0],
                dst_ref=comm_ref.at[k],
                send_sem=send_sems.at[k - 1],
                recv_sem=recv_sems.at[k - 1],
                device_id=((my + k) % N_DEV,),
                device_id_type=pl.DeviceIdType.MESH,
            )
            rdma.start()
            rdmas.append(rdma)
        for rdma in rdmas:
            rdma.wait_recv()

        stats = comm_ref[:, :, :]
        m_all = stats[:, 0, :]
        s_all = stats[:, 1, :]
        gmax = jnp.max(m_all, axis=0, keepdims=True)
        gsum = jnp.sum(s_all * jnp.exp(m_all - gmax), axis=0,
                       keepdims=True)
        c = jnp.transpose(gmax + jnp.log(gsum))

        out_ref[:, :] = jnp.exp(xv - c)

        for rdma in rdmas:
            rdma.wait_send()

    return pl.pallas_call(
        body,
        out_shape=jax.ShapeDtypeStruct((m_rows, n_cols), jnp.float32),
        in_specs=[pl.BlockSpec(memory_space=pltpu.VMEM)],
        out_specs=pl.BlockSpec(memory_space=pltpu.VMEM),
        scratch_shapes=[
            pltpu.VMEM((N_DEV, 2, m_rows), jnp.float32),
            pltpu.SemaphoreType.DMA((N_DEV - 1,)),
            pltpu.SemaphoreType.DMA((N_DEV - 1,)),
        ],
        compiler_params=pltpu.CompilerParams(collective_id=0),
    )(x)
